# baseline (speedup 1.0000x reference)
import functools

import jax
import jax.numpy as jnp
from jax import lax
from jax.experimental import pallas as pl
from jax.experimental.pallas import tpu as pltpu
from jax.experimental.pallas import tpu_sc as plsc


def _matmul_body(x_ref, w_ref, z_ref):
    x = x_ref[0].astype(jnp.bfloat16)
    z_ref[0] = jnp.dot(x, w_ref[...], preferred_element_type=jnp.float32)


def _dense_z(X, Wt, tb):
    B, T, I = X.shape
    EJ = Wt.shape[1]
    grid = (B, T // tb)
    return pl.pallas_call(
        _matmul_body,
        grid=grid,
        in_specs=[
            pl.BlockSpec((1, tb, I), lambda b, t: (b, t, 0)),
            pl.BlockSpec((I, EJ), lambda b, t: (0, 0)),
        ],
        out_specs=pl.BlockSpec((1, tb, EJ), lambda b, t: (b, t, 0)),
        out_shape=jax.ShapeDtypeStruct((B, T, EJ), jnp.float32),
    )(X, Wt)


def _make_gather(n_rows, J, T, E, K, n_tiles, chunk=128):
    rows_per_tile = n_rows // n_tiles
    n_chunks = rows_per_tile // chunk
    mesh = plsc.VectorSubcoreMesh(core_axis_name="c", subcore_axis_name="s")
    info = plsc.get_sparse_core_info()
    nc = info.num_cores

    @functools.partial(
        pl.kernel,
        mesh=mesh,
        out_type=jax.ShapeDtypeStruct((n_rows, J), jnp.float32),
        scratch_types=[
            pltpu.VMEM((n_chunks, chunk), jnp.int32),
            pltpu.VMEM((chunk, J), jnp.float32),
            pltpu.SemaphoreType.DMA,
        ],
    )
    def gather(z_hbm, idx_hbm, out_hbm, idx_v, rows_v, sem):
        wid = lax.axis_index("s") * nc + lax.axis_index("c")
        pltpu.sync_copy(idx_hbm.at[wid], idx_v)
        pe = (wid * rows_per_tile) // K
        b = pe // E
        e = pe - b * E
        off = (b * T * E + e).astype(jnp.int32)
        e32 = jnp.int32(E)
        for c in range(n_chunks):
            for i in range(chunk // 16):
                sl = (c, pl.ds(i * 16, 16))
                idx_v[sl] = idx_v[sl] * e32 + off
        base = wid * rows_per_tile
        for c in range(n_chunks):
            pltpu.async_copy(z_hbm.at[idx_v.at[c]], rows_v, sem).wait()
            pltpu.sync_copy(rows_v, out_hbm.at[pl.ds(base + c * chunk, chunk)])

    return gather


def kernel(X, ind, W):
    B, T, I = X.shape
    E, _, J = W.shape
    K = ind.shape[2]
    info = plsc.get_sparse_core_info()
    n_tiles = info.num_cores * info.num_subcores
    assert B * E == n_tiles and K % 128 == 0

    Wt = W.transpose(1, 0, 2).reshape(I, E * J).astype(jnp.bfloat16)
    Z = _dense_z(X, Wt, tb=2048)
    z_flat = Z.reshape(B * T * E, J)
    idx = ind.astype(jnp.int32).reshape(n_tiles, K // 128, 128)
    y_flat = _make_gather(B * E * K, J, T, E, K, n_tiles)(z_flat, idx)
    return y_flat.reshape(B, E, K, J)


# speedup vs baseline: 1.6866x; 1.6866x over previous
import functools

import jax
import jax.numpy as jnp
from jax import lax
from jax.experimental import pallas as pl
from jax.experimental.pallas import tpu as pltpu
from jax.experimental.pallas import tpu_sc as plsc


def _matmul_body(e_count, x_ref, w_ref, z_ref):
    x = x_ref[0].astype(jnp.bfloat16)
    z = jnp.dot(x, w_ref[...], preferred_element_type=jnp.float32)
    j = z.shape[1] // e_count
    for e in range(e_count):
        z_ref[0, e] = z[:, e * j:(e + 1) * j]


def _dense_z(X, Wt, E, J, tb):
    B, T, I = X.shape
    grid = (B, T // tb)
    return pl.pallas_call(
        functools.partial(_matmul_body, E),
        grid=grid,
        in_specs=[
            pl.BlockSpec((1, tb, I), lambda b, t: (b, t, 0)),
            pl.BlockSpec((I, E * J), lambda b, t: (0, 0)),
        ],
        out_specs=pl.BlockSpec((1, E, tb, J), lambda b, t: (b, 0, t, 0)),
        out_shape=jax.ShapeDtypeStruct((B, E, T, J), jnp.float32),
    )(X, Wt)


def _make_gather(n_rows, J, T, K, n_tiles, chunk=128):
    rows_per_tile = n_rows // n_tiles
    n_chunks = rows_per_tile // chunk
    mesh = plsc.VectorSubcoreMesh(core_axis_name="c", subcore_axis_name="s")
    info = plsc.get_sparse_core_info()
    nc = info.num_cores

    @functools.partial(
        pl.kernel,
        mesh=mesh,
        out_type=jax.ShapeDtypeStruct((n_rows, J), jnp.float32),
        scratch_types=[
            pltpu.VMEM((n_chunks, chunk), jnp.int32),
            pltpu.VMEM((chunk, J), jnp.float32),
            pltpu.SemaphoreType.DMA,
        ],
    )
    def gather(z_hbm, idx_hbm, out_hbm, idx_v, rows_v, sem):
        wid = lax.axis_index("s") * nc + lax.axis_index("c")
        pltpu.sync_copy(idx_hbm.at[wid], idx_v)
        off = (((wid * rows_per_tile) // K) * T).astype(jnp.int32)
        for c in range(n_chunks):
            for i in range(chunk // 16):
                sl = (c, pl.ds(i * 16, 16))
                idx_v[sl] = idx_v[sl] + off
        base = wid * rows_per_tile
        for c in range(n_chunks):
            pltpu.async_copy(z_hbm.at[idx_v.at[c]], rows_v, sem).wait()
            pltpu.sync_copy(rows_v, out_hbm.at[pl.ds(base + c * chunk, chunk)])

    return gather


def kernel(X, ind, W):
    B, T, I = X.shape
    E, _, J = W.shape
    K = ind.shape[2]
    info = plsc.get_sparse_core_info()
    n_tiles = info.num_cores * info.num_subcores
    assert B * E == n_tiles and K % 128 == 0

    Wt = W.transpose(1, 0, 2).reshape(I, E * J).astype(jnp.bfloat16)
    Z = _dense_z(X, Wt, E, J, tb=1024)
    z_flat = Z.reshape(B * E * T, J)
    idx = ind.astype(jnp.int32).reshape(n_tiles, K // 128, 128)
    y_flat = _make_gather(B * E * K, J, T, K, n_tiles)(z_flat, idx)
    return y_flat.reshape(B, E, K, J)


# speedup vs baseline: 1.8332x; 1.0869x over previous
import functools

import jax
import jax.numpy as jnp
from jax import lax
from jax.experimental import pallas as pl
from jax.experimental.pallas import tpu as pltpu
from jax.experimental.pallas import tpu_sc as plsc


def _matmul_body(e_count, x_ref, w_ref, z_ref):
    x = x_ref[0].astype(jnp.bfloat16)
    z = jnp.dot(x, w_ref[...], preferred_element_type=jnp.float32)
    j = z.shape[1] // e_count
    for e in range(e_count):
        z_ref[0, e] = z[:, e * j:(e + 1) * j]


def _dense_z(X, Wt, E, J, tb):
    B, T, I = X.shape
    grid = (B, T // tb)
    return pl.pallas_call(
        functools.partial(_matmul_body, E),
        grid=grid,
        in_specs=[
            pl.BlockSpec((1, tb, I), lambda b, t: (b, t, 0)),
            pl.BlockSpec((I, E * J), lambda b, t: (0, 0)),
        ],
        out_specs=pl.BlockSpec((1, E, tb, J), lambda b, t: (b, 0, t, 0)),
        out_shape=jax.ShapeDtypeStruct((B, E, T, J), jnp.float32),
    )(X, Wt)


def _make_gather(n_rows, J, T, K, n_tiles, chunk=128):
    rows_per_tile = n_rows // n_tiles
    n_chunks = rows_per_tile // chunk
    mesh = plsc.VectorSubcoreMesh(core_axis_name="c", subcore_axis_name="s")
    info = plsc.get_sparse_core_info()
    nc = info.num_cores

    @functools.partial(
        pl.kernel,
        mesh=mesh,
        out_type=jax.ShapeDtypeStruct((n_rows, J), jnp.float32),
        scratch_types=[
            pltpu.VMEM((n_chunks, chunk), jnp.int32),
            pltpu.VMEM((2, chunk, J), jnp.float32),
            [pltpu.SemaphoreType.DMA] * 2,
            [pltpu.SemaphoreType.DMA] * 2,
        ],
    )
    def gather(z_hbm, idx_hbm, out_hbm, idx_v, rows_v, gsem, ssem):
        wid = lax.axis_index("s") * nc + lax.axis_index("c")
        pltpu.sync_copy(idx_hbm.at[wid], idx_v)
        off = (((wid * rows_per_tile) // K) * T).astype(jnp.int32)
        for c in range(n_chunks):
            for i in range(chunk // 16):
                sl = (c, pl.ds(i * 16, 16))
                idx_v[sl] = idx_v[sl] + off
        base = wid * rows_per_tile
        gathers = [None] * n_chunks
        stores = [None] * n_chunks
        gathers[0] = pltpu.async_copy(
            z_hbm.at[idx_v.at[0]], rows_v.at[0], gsem[0])
        for c in range(n_chunks):
            b = c % 2
            if c + 1 < n_chunks:
                if c >= 1 and stores[c - 1] is not None:
                    stores[c - 1].wait()
                gathers[c + 1] = pltpu.async_copy(
                    z_hbm.at[idx_v.at[c + 1]], rows_v.at[1 - b], gsem[1 - b])
            gathers[c].wait()
            stores[c] = pltpu.make_async_copy(
                rows_v.at[b], out_hbm.at[pl.ds(base + c * chunk, chunk)],
                ssem[b])
            stores[c].start()
        stores[n_chunks - 1].wait()
        if n_chunks >= 2:
            stores[n_chunks - 2].wait()

    return gather


def kernel(X, ind, W):
    B, T, I = X.shape
    E, _, J = W.shape
    K = ind.shape[2]
    info = plsc.get_sparse_core_info()
    n_tiles = info.num_cores * info.num_subcores
    assert B * E == n_tiles and K % 128 == 0

    Wt = W.transpose(1, 0, 2).reshape(I, E * J).astype(jnp.bfloat16)
    Z = _dense_z(X, Wt, E, J, tb=2048)
    z_flat = Z.reshape(B * E * T, J)
    idx = ind.astype(jnp.int32).reshape(n_tiles, K // 128, 128)
    y_flat = _make_gather(B * E * K, J, T, K, n_tiles)(z_flat, idx)
    return y_flat.reshape(B, E, K, J)


# speedup vs baseline: 1.8476x; 1.0079x over previous
import functools

import jax
import jax.numpy as jnp
from jax import lax
from jax.experimental import pallas as pl
from jax.experimental.pallas import tpu as pltpu
from jax.experimental.pallas import tpu_sc as plsc


def _matmul_body(e_count, x_ref, w_ref, z_ref):
    x = x_ref[0].astype(jnp.bfloat16)
    z = jnp.dot(x, w_ref[...], preferred_element_type=jnp.float32)
    j = z.shape[1] // e_count
    for e in range(e_count):
        z_ref[0, e] = z[:, e * j:(e + 1) * j]


def _dense_z(X, Wt, E, J, tb):
    B, T, I = X.shape
    grid = (B, T // tb)
    return pl.pallas_call(
        functools.partial(_matmul_body, E),
        grid=grid,
        in_specs=[
            pl.BlockSpec((1, tb, I), lambda b, t: (b, t, 0)),
            pl.BlockSpec((I, E * J), lambda b, t: (0, 0)),
        ],
        out_specs=pl.BlockSpec((1, E, tb, J), lambda b, t: (b, 0, t, 0)),
        out_shape=jax.ShapeDtypeStruct((B, E, T, J), jnp.float32),
    )(X, Wt)


def _make_gather(n_rows, J, T, K, n_tiles, chunk=128):
    rows_per_tile = n_rows // n_tiles
    n_chunks = rows_per_tile // chunk
    mesh = plsc.VectorSubcoreMesh(core_axis_name="c", subcore_axis_name="s")
    info = plsc.get_sparse_core_info()
    nc = info.num_cores

    nbuf = 4
    ahead = 3

    @functools.partial(
        pl.kernel,
        mesh=mesh,
        out_type=jax.ShapeDtypeStruct((n_rows, J), jnp.float32),
        scratch_types=[
            pltpu.VMEM((n_chunks, chunk), jnp.int32),
            pltpu.VMEM((nbuf, chunk, J), jnp.float32),
            [pltpu.SemaphoreType.DMA] * nbuf,
            [pltpu.SemaphoreType.DMA] * nbuf,
        ],
    )
    def gather(z_hbm, idx_hbm, out_hbm, idx_v, rows_v, gsem, ssem):
        wid = lax.axis_index("s") * nc + lax.axis_index("c")
        pltpu.sync_copy(idx_hbm.at[wid], idx_v)
        off = (((wid * rows_per_tile) // K) * T).astype(jnp.int32)
        for c in range(n_chunks):
            for i in range(chunk // 16):
                sl = (c, pl.ds(i * 16, 16))
                idx_v[sl] = idx_v[sl] + off
        base = wid * rows_per_tile

        def issue(c):
            return pltpu.async_copy(
                z_hbm.at[idx_v.at[c]], rows_v.at[c % nbuf], gsem[c % nbuf])

        gathers = [None] * n_chunks
        stores = [None] * n_chunks
        store_waited = set()
        for c in range(min(ahead, n_chunks)):
            gathers[c] = issue(c)
        for c in range(n_chunks):
            nxt = c + ahead
            if nxt < n_chunks:
                prev = nxt - nbuf
                if prev >= 0:
                    stores[prev].wait()
                    store_waited.add(prev)
                gathers[nxt] = issue(nxt)
            gathers[c].wait()
            stores[c] = pltpu.make_async_copy(
                rows_v.at[c % nbuf],
                out_hbm.at[pl.ds(base + c * chunk, chunk)],
                ssem[c % nbuf])
            stores[c].start()
        for c in range(n_chunks):
            if c not in store_waited:
                stores[c].wait()

    return gather


def kernel(X, ind, W):
    B, T, I = X.shape
    E, _, J = W.shape
    K = ind.shape[2]
    info = plsc.get_sparse_core_info()
    n_tiles = info.num_cores * info.num_subcores
    assert B * E == n_tiles and K % 128 == 0

    Wt = W.transpose(1, 0, 2).reshape(I, E * J).astype(jnp.bfloat16)
    Z = _dense_z(X, Wt, E, J, tb=2048)
    z_flat = Z.reshape(B * E * T, J)
    idx = ind.astype(jnp.int32).reshape(n_tiles, K // 128, 128)
    y_flat = _make_gather(B * E * K, J, T, K, n_tiles)(z_flat, idx)
    return y_flat.reshape(B, E, K, J)
